# baseline (speedup 1.0000x reference)
import functools

import jax
import jax.numpy as jnp
from jax.experimental import pallas as pl
from jax.experimental.pallas import tpu as pltpu
from jax.experimental.pallas import tpu_sc as plsc

L = 4096
N = 8
C = 3
D_IN = 128
D_OUT = 256
K = 16
L2 = 1024

_NC, _NS = 2, 16
_NW = _NC * _NS
_QPW = (L2 * N) // _NW
_CH = 4
_NCHUNK = _QPW // _CH
_ROWS = _CH * K


def _pool_body(cidx_ref, table_ref, out_ref, idx_v, buf0, buf1, ob, sem0, sem1):
    wid = jax.lax.axis_index("s") * _NC + jax.lax.axis_index("c")
    pltpu.sync_copy(cidx_ref.at[wid], idx_v)
    bufs = (buf0, buf1)
    sems = (sem0, sem1)

    def fire(c, b):
        pltpu.make_async_copy(table_ref.at[idx_v.at[c]], bufs[b], sems[b]).start()

    def wait(b):
        pltpu.make_async_copy(table_ref.at[idx_v.at[0]], bufs[b], sems[b]).wait()

    def compute(c, b):
        buf = bufs[b]

        def dstep(d, carry):
            off = d * 16
            for qq in range(_CH):
                m = buf[qq * K, pl.ds(off, 16)]
                for r in range(1, K):
                    m = jnp.maximum(m, buf[qq * K + r, pl.ds(off, 16)])
                ob[qq, pl.ds(off, 16)] = m
            return carry

        jax.lax.fori_loop(0, D_OUT // 16, dstep, 0)
        pltpu.sync_copy(ob, out_ref.at[pl.ds(wid * _QPW + c * _CH, _CH)])

    fire(0, 0)
    fire(1, 1)

    def step(i, carry):
        c = i * 2
        wait(0)
        compute(c, 0)

        @pl.when(c + 2 < _NCHUNK)
        def _():
            fire(c + 2, 0)

        wait(1)
        compute(c + 1, 1)

        @pl.when(c + 3 < _NCHUNK)
        def _():
            fire(c + 3, 1)

        return carry

    jax.lax.fori_loop(0, _NCHUNK // 2, step, 0)


def _pool_pallas(cidx3, table):
    f = functools.partial(
        pl.kernel,
        mesh=plsc.VectorSubcoreMesh(core_axis_name="c", subcore_axis_name="s"),
        out_type=jax.ShapeDtypeStruct((L2 * N, D_OUT), jnp.float32),
        scratch_types=[
            pltpu.VMEM((_NCHUNK, _ROWS), jnp.int32),
            pltpu.VMEM((_ROWS, D_OUT), jnp.float32),
            pltpu.VMEM((_ROWS, D_OUT), jnp.float32),
            pltpu.VMEM((_CH, D_OUT), jnp.float32),
            pltpu.SemaphoreType.DMA,
            pltpu.SemaphoreType.DMA,
        ],
    )(_pool_body)
    return f(cidx3, table)


def _mlp_body(x_ref, w1_ref, g1_ref, b1_ref, w2_ref, g2_ref, b2_ref, o_ref):
    x = x_ref[...]
    h = jnp.dot(x, w1_ref[...].T, preferred_element_type=jnp.float32)
    mu = jnp.mean(h, axis=-1, keepdims=True)
    var = jnp.mean((h - mu) ** 2, axis=-1, keepdims=True)
    h = (h - mu) / jnp.sqrt(var + 1e-5) * g1_ref[...] + b1_ref[...]
    h = jnp.dot(h, w2_ref[...].T, preferred_element_type=jnp.float32)
    mu = jnp.mean(h, axis=-1, keepdims=True)
    var = jnp.mean((h - mu) ** 2, axis=-1, keepdims=True)
    h = (h - mu) / jnp.sqrt(var + 1e-5) * g2_ref[...] + b2_ref[...]
    o_ref[...] = jnp.maximum(h, 0.0)


def _mlp_pallas(x2d, W1, g1, b1, W2, g2, b2):
    R = x2d.shape[0]
    BR = 1024
    grid = (R // BR,)
    return pl.pallas_call(
        _mlp_body,
        grid=grid,
        in_specs=[
            pl.BlockSpec((BR, D_IN), lambda i: (i, 0)),
            pl.BlockSpec((D_OUT, D_IN), lambda i: (0, 0)),
            pl.BlockSpec((D_OUT,), lambda i: (0,)),
            pl.BlockSpec((D_OUT,), lambda i: (0,)),
            pl.BlockSpec((D_OUT, D_OUT), lambda i: (0, 0)),
            pl.BlockSpec((D_OUT,), lambda i: (0,)),
            pl.BlockSpec((D_OUT,), lambda i: (0,)),
        ],
        out_specs=pl.BlockSpec((BR, D_OUT), lambda i: (i, 0)),
        out_shape=jax.ShapeDtypeStruct((R, D_OUT), jnp.float32),
    )(x2d, W1, g1, b1, W2, g2, b2)


def kernel(coords, features, W1, g1, b1, W2, g2, b2):
    keep = jax.random.permutation(jax.random.key(42), L)[:L2]
    keep_l = jnp.repeat(keep, N)
    keep_n = jnp.tile(jnp.arange(N), L2)
    keep_coords = coords[keep_l, keep_n].reshape(L2, N, C)

    x2d = features[:L2].reshape(L2 * N, D_IN)
    feats_sub = _mlp_pallas(x2d, W1, g1, b1, W2, g2, b2).reshape(L2, N, D_OUT)

    c1 = jnp.swapaxes(coords, 0, 1)
    c2 = jnp.swapaxes(keep_coords, 0, 1)
    d2 = (jnp.sum(c2 ** 2, axis=-1)[:, :, None]
          + jnp.sum(c1 ** 2, axis=-1)[:, None, :]
          - 2.0 * jnp.einsum('nqc,nlc->nql', c2, c1))
    _, local = jax.lax.top_k(-d2, K)
    glob = local + jnp.arange(N)[:, None, None] * L
    clusters = jnp.mod(glob, L2)
    clusters = jnp.transpose(clusters, (2, 1, 0))
    idx0 = clusters.reshape(-1)
    idx1 = jnp.broadcast_to(jnp.arange(N)[None, None, :], (K, L2, N)).reshape(-1)

    cidx = clusters.astype(jnp.int32) * N + jnp.arange(N, dtype=jnp.int32)[None, None, :]
    cidx3 = jnp.transpose(cidx, (1, 2, 0)).reshape(_NW, _NCHUNK, _ROWS)
    table = feats_sub.reshape(L2 * N, D_OUT)
    pool = _pool_pallas(cidx3, table).reshape(L2, N, D_OUT)
    return keep_coords, pool, (idx0, idx1), (keep_l, keep_n)


# speedup vs baseline: 2.6432x; 2.6432x over previous
import functools

import jax
import jax.numpy as jnp
from jax.experimental import pallas as pl
from jax.experimental.pallas import tpu as pltpu
from jax.experimental.pallas import tpu_sc as plsc

L = 4096
N = 8
C = 3
D_IN = 128
D_OUT = 256
K = 16
L2 = 1024

_NC, _NS = 2, 16
_NW = _NC * _NS
_QPW = (L2 * N) // _NW
_CH = 4
_NCHUNK = _QPW // _CH
_ROWS = _CH * K


_BQ = 256
_NG = 128
_GS = 32


def _theta_body(gm_ref, th_ref):
    gm = gm_ref[...]
    lanes = jax.lax.broadcasted_iota(jnp.int32, (_BQ, _NG), 1).astype(jnp.float32)
    for t in range(K - 1):
        m = jnp.min(gm, axis=1, keepdims=True)
        gid = jnp.min(jnp.where(gm == m, lanes, 1e9), axis=1, keepdims=True)
        gm = jnp.where(lanes == gid, jnp.inf, gm)
    th = jnp.min(gm, axis=1, keepdims=True)
    th_ref[...] = jnp.broadcast_to(th, (_BQ, K))


def _theta_pallas(gm):
    return pl.pallas_call(
        _theta_body,
        grid=((N * L2) // _BQ,),
        in_specs=[pl.BlockSpec((_BQ, _NG), lambda i: (i, 0))],
        out_specs=pl.BlockSpec((_BQ, K), lambda i: (i, 0)),
        out_shape=jax.ShapeDtypeStruct((N * L2, K), jnp.float32),
    )(gm)


_QPW_B = (L2 * N) // _NW


def _topk_body(d_ref, th_ref, out_ref, th_v, dv0, dv1, ob, tvb, tib,
               sem0, sem1):
    wid = jax.lax.axis_index("s") * _NC + jax.lax.axis_index("c")
    base = wid * _QPW_B
    pltpu.sync_copy(th_ref.at[pl.ds(base, _QPW_B)], th_v)
    dbufs = (dv0, dv1)
    sems = (sem0, sem1)

    def fire(j, b):
        pltpu.make_async_copy(d_ref.at[base + j], dbufs[b], sems[b]).start()

    def wait(b):
        pltpu.make_async_copy(d_ref.at[base], dbufs[b], sems[b]).wait()

    iota16 = jax.lax.iota(jnp.int32, 16)

    def compute(j, b):
        dv = dbufs[b]
        th = th_v[j, pl.ds(0, 16)]
        nn = jax.lax.shift_right_logical(base + j, 10)
        tvb[pl.ds(0, 16)] = jnp.full((16,), 0x7F800000, dtype=jnp.int32)
        tib[pl.ds(0, 16)] = jnp.zeros((16,), dtype=jnp.int32)

        def chunk(c, carry):
            val = dv[pl.ds(c * 16, 16)]
            hit = val <= th
            cnt = plsc.all_reduce_population_count(hit)

            @pl.when(cnt[0] > 0)
            def _():
                idxv = iota16 + c * 16
                bits = plsc.bitcast(val, jnp.int32)
                key = jnp.where(val < 0.0,
                                jnp.bitwise_xor(bits, 0x7FFFFFFF), bits)
                sv, si = plsc.sort_key_val(key, idxv, descending=True)
                tvv = tvb[pl.ds(0, 16)]
                tii = tib[pl.ds(0, 16)]
                take = sv < tvv
                mv = jnp.where(take, sv, tvv)
                mi = jnp.where(take, si, tii)
                tv2, ti2 = plsc.sort_key_val(mv, mi)
                tvb[pl.ds(0, 16)] = tv2
                tib[pl.ds(0, 16)] = ti2

            return carry

        jax.lax.fori_loop(0, L // 16, chunk, 0)
        ti = tib[pl.ds(0, 16)]
        ob[j, pl.ds(0, 16)] = (
            jnp.left_shift(jnp.bitwise_and(ti, L2 - 1), 3) + nn)

    fire(0, 0)
    fire(1, 1)

    def step(i, carry):
        j = i * 2
        wait(0)
        compute(j, 0)

        @pl.when(j + 2 < _QPW_B)
        def _():
            fire(j + 2, 0)

        wait(1)
        compute(j + 1, 1)

        @pl.when(j + 3 < _QPW_B)
        def _():
            fire(j + 3, 1)

        return carry

    jax.lax.fori_loop(0, _QPW_B // 2, step, 0)
    pltpu.sync_copy(ob, out_ref.at[pl.ds(base, _QPW_B)])


def _topk_pallas(d2f, thf):
    f = functools.partial(
        pl.kernel,
        mesh=plsc.VectorSubcoreMesh(core_axis_name="c", subcore_axis_name="s"),
        compiler_params=pltpu.CompilerParams(needs_layout_passes=False),
        out_type=jax.ShapeDtypeStruct((L2 * N, K), jnp.int32),
        scratch_types=[
            pltpu.VMEM((_QPW_B, K), jnp.float32),
            pltpu.VMEM((L,), jnp.float32),
            pltpu.VMEM((L,), jnp.float32),
            pltpu.VMEM((_QPW_B, K), jnp.int32),
            pltpu.VMEM((K,), jnp.int32),
            pltpu.VMEM((K,), jnp.int32),
            pltpu.SemaphoreType.DMA,
            pltpu.SemaphoreType.DMA,
        ],
    )(_topk_body)
    return f(d2f, thf)


def _pool_body(cidx_ref, table_ref, out_ref, idx_v, buf0, buf1, ob, sem0, sem1):
    wid = jax.lax.axis_index("s") * _NC + jax.lax.axis_index("c")
    pltpu.sync_copy(cidx_ref.at[wid], idx_v)
    bufs = (buf0, buf1)
    sems = (sem0, sem1)

    def fire(c, b):
        pltpu.make_async_copy(table_ref.at[idx_v.at[c]], bufs[b], sems[b]).start()

    def wait(b):
        pltpu.make_async_copy(table_ref.at[idx_v.at[0]], bufs[b], sems[b]).wait()

    def compute(c, b):
        buf = bufs[b]

        def dstep(d, carry):
            off = d * 16
            for qq in range(_CH):
                m = buf[qq * K, pl.ds(off, 16)]
                for r in range(1, K):
                    m = jnp.maximum(m, buf[qq * K + r, pl.ds(off, 16)])
                ob[qq, pl.ds(off, 16)] = m
            return carry

        jax.lax.fori_loop(0, D_OUT // 16, dstep, 0)
        pltpu.sync_copy(ob, out_ref.at[pl.ds(wid * _QPW + c * _CH, _CH)])

    fire(0, 0)
    fire(1, 1)

    def step(i, carry):
        c = i * 2
        wait(0)
        compute(c, 0)

        @pl.when(c + 2 < _NCHUNK)
        def _():
            fire(c + 2, 0)

        wait(1)
        compute(c + 1, 1)

        @pl.when(c + 3 < _NCHUNK)
        def _():
            fire(c + 3, 1)

        return carry

    jax.lax.fori_loop(0, _NCHUNK // 2, step, 0)


def _pool_pallas(cidx3, table):
    f = functools.partial(
        pl.kernel,
        mesh=plsc.VectorSubcoreMesh(core_axis_name="c", subcore_axis_name="s"),
        out_type=jax.ShapeDtypeStruct((L2 * N, D_OUT), jnp.float32),
        scratch_types=[
            pltpu.VMEM((_NCHUNK, _ROWS), jnp.int32),
            pltpu.VMEM((_ROWS, D_OUT), jnp.float32),
            pltpu.VMEM((_ROWS, D_OUT), jnp.float32),
            pltpu.VMEM((_CH, D_OUT), jnp.float32),
            pltpu.SemaphoreType.DMA,
            pltpu.SemaphoreType.DMA,
        ],
    )(_pool_body)
    return f(cidx3, table)


def _mlp_body(x_ref, w1_ref, g1_ref, b1_ref, w2_ref, g2_ref, b2_ref, o_ref):
    x = x_ref[...]
    h = jnp.dot(x, w1_ref[...].T, preferred_element_type=jnp.float32)
    mu = jnp.mean(h, axis=-1, keepdims=True)
    var = jnp.mean((h - mu) ** 2, axis=-1, keepdims=True)
    h = (h - mu) / jnp.sqrt(var + 1e-5) * g1_ref[...] + b1_ref[...]
    h = jnp.dot(h, w2_ref[...].T, preferred_element_type=jnp.float32)
    mu = jnp.mean(h, axis=-1, keepdims=True)
    var = jnp.mean((h - mu) ** 2, axis=-1, keepdims=True)
    h = (h - mu) / jnp.sqrt(var + 1e-5) * g2_ref[...] + b2_ref[...]
    o_ref[...] = jnp.maximum(h, 0.0)


def _mlp_pallas(x2d, W1, g1, b1, W2, g2, b2):
    R = x2d.shape[0]
    BR = 1024
    grid = (R // BR,)
    return pl.pallas_call(
        _mlp_body,
        grid=grid,
        in_specs=[
            pl.BlockSpec((BR, D_IN), lambda i: (i, 0)),
            pl.BlockSpec((D_OUT, D_IN), lambda i: (0, 0)),
            pl.BlockSpec((D_OUT,), lambda i: (0,)),
            pl.BlockSpec((D_OUT,), lambda i: (0,)),
            pl.BlockSpec((D_OUT, D_OUT), lambda i: (0, 0)),
            pl.BlockSpec((D_OUT,), lambda i: (0,)),
            pl.BlockSpec((D_OUT,), lambda i: (0,)),
        ],
        out_specs=pl.BlockSpec((BR, D_OUT), lambda i: (i, 0)),
        out_shape=jax.ShapeDtypeStruct((R, D_OUT), jnp.float32),
    )(x2d, W1, g1, b1, W2, g2, b2)


def kernel(coords, features, W1, g1, b1, W2, g2, b2):
    keep = jax.random.permutation(jax.random.key(42), L)[:L2]
    keep_l = jnp.repeat(keep, N)
    keep_n = jnp.tile(jnp.arange(N), L2)
    keep_coords = coords[keep_l, keep_n].reshape(L2, N, C)

    x2d = features[:L2].reshape(L2 * N, D_IN)
    feats_sub = _mlp_pallas(x2d, W1, g1, b1, W2, g2, b2).reshape(L2, N, D_OUT)

    c1s = jnp.swapaxes(coords, 0, 1)
    c2s = jnp.swapaxes(keep_coords, 0, 1)
    d2 = (jnp.sum(c2s ** 2, axis=-1)[:, :, None]
          + jnp.sum(c1s ** 2, axis=-1)[:, None, :]
          - 2.0 * jnp.einsum('nqc,nlc->nql', c2s, c1s))
    gm = jnp.min(d2.reshape(N, L2, _NG, _GS), axis=-1)
    th = _theta_pallas(gm.reshape(N * L2, _NG))
    out_nq = _topk_pallas(d2.reshape(N * L2, L), th)
    out_qn = jnp.transpose(out_nq.reshape(N, L2, K),
                           (1, 0, 2)).reshape(L2 * N, K)

    idx0 = jnp.transpose(jnp.right_shift(out_qn, 3).reshape(L2, N, K),
                         (2, 0, 1)).reshape(-1)
    idx1 = jnp.broadcast_to(jnp.arange(N)[None, None, :], (K, L2, N)).reshape(-1)

    cidx3 = out_qn.reshape(_NW, _NCHUNK, _ROWS)
    table = feats_sub.reshape(L2 * N, D_OUT)
    pool = _pool_pallas(cidx3, table).reshape(L2, N, D_OUT)
    return keep_coords, pool, (idx0, idx1), (keep_l, keep_n)


# speedup vs baseline: 6.8091x; 2.5761x over previous
import functools

import jax
import jax.numpy as jnp
from jax.experimental import pallas as pl
from jax.experimental.pallas import tpu as pltpu
from jax.experimental.pallas import tpu_sc as plsc

L = 4096
N = 8
C = 3
D_IN = 128
D_OUT = 256
K = 16
L2 = 1024

_NC, _NS = 2, 16
_NW = _NC * _NS
_QPW = (L2 * N) // _NW
_CH = 4
_NCHUNK = _QPW // _CH
_ROWS = _CH * K


_BQ = 256
_NG = 128
_GS = 32


def _theta_body(gm_ref, th_ref):
    gm = gm_ref[...]
    lanes = jax.lax.broadcasted_iota(jnp.int32, (_BQ, _NG), 1).astype(jnp.float32)
    for t in range(K - 1):
        m = jnp.min(gm, axis=1, keepdims=True)
        gid = jnp.min(jnp.where(gm == m, lanes, 1e9), axis=1, keepdims=True)
        gm = jnp.where(lanes == gid, jnp.inf, gm)
    th = jnp.min(gm, axis=1, keepdims=True)
    th_ref[...] = jnp.broadcast_to(th, (_BQ, K))


def _theta_pallas(gm):
    return pl.pallas_call(
        _theta_body,
        grid=((N * L2) // _BQ,),
        in_specs=[pl.BlockSpec((_BQ, _NG), lambda i: (i, 0))],
        out_specs=pl.BlockSpec((_BQ, K), lambda i: (i, 0)),
        out_shape=jax.ShapeDtypeStruct((N * L2, K), jnp.float32),
    )(gm)


_QPW_B = (L2 * N) // _NW


def _topk_body(d_ref, th_ref, out_ref, th_v, dv0, dv1, ob, tvb, tib,
               sem0, sem1):
    wid = jax.lax.axis_index("s") * _NC + jax.lax.axis_index("c")
    base = wid * _QPW_B
    pltpu.sync_copy(th_ref.at[pl.ds(base, _QPW_B)], th_v)
    dbufs = (dv0, dv1)
    sems = (sem0, sem1)

    def fire(j, b):
        pltpu.make_async_copy(d_ref.at[base + j], dbufs[b], sems[b]).start()

    def wait(b):
        pltpu.make_async_copy(d_ref.at[base], dbufs[b], sems[b]).wait()

    iota16 = jax.lax.iota(jnp.int32, 16)

    def compute(j, b):
        dv = dbufs[b]
        th = th_v[j, pl.ds(0, 16)]
        nn = jax.lax.shift_right_logical(base + j, 10)
        tvb[pl.ds(0, 16)] = jnp.full((16,), 0x7F800000, dtype=jnp.int32)
        tib[pl.ds(0, 16)] = jnp.zeros((16,), dtype=jnp.int32)

        def merge(val, idxv):
            bits = plsc.bitcast(val, jnp.int32)
            key = jnp.where(val < 0.0,
                            jnp.bitwise_xor(bits, 0x7FFFFFFF), bits)
            sv, si = plsc.sort_key_val(key, idxv, descending=True)
            tvv = tvb[pl.ds(0, 16)]
            tii = tib[pl.ds(0, 16)]
            take = sv < tvv
            mv = jnp.where(take, sv, tvv)
            mi = jnp.where(take, si, tii)
            tv2, ti2 = plsc.sort_key_val(mv, mi)
            tvb[pl.ds(0, 16)] = tv2
            tib[pl.ds(0, 16)] = ti2

        _U = 4

        def chunk(c4, carry):
            cbase = c4 * _U
            vals = [dv[pl.ds((cbase + u) * 16, 16)] for u in range(_U)]
            hits = [v <= th for v in vals]
            anyv = jnp.logical_or(jnp.logical_or(hits[0], hits[1]),
                                  jnp.logical_or(hits[2], hits[3]))
            cnt = plsc.all_reduce_population_count(anyv)

            @pl.when(cnt[0] > 0)
            def _():
                for u in range(_U):
                    cntu = plsc.all_reduce_population_count(hits[u])

                    @pl.when(cntu[0] > 0)
                    def _(u=u):
                        merge(vals[u], iota16 + (cbase + u) * 16)

            return carry

        jax.lax.fori_loop(0, L // 16 // _U, chunk, 0)
        ti = tib[pl.ds(0, 16)]
        ob[j, pl.ds(0, 16)] = (
            jnp.left_shift(jnp.bitwise_and(ti, L2 - 1), 3) + nn)

    fire(0, 0)
    fire(1, 1)

    def step(i, carry):
        j = i * 2
        wait(0)
        compute(j, 0)

        @pl.when(j + 2 < _QPW_B)
        def _():
            fire(j + 2, 0)

        wait(1)
        compute(j + 1, 1)

        @pl.when(j + 3 < _QPW_B)
        def _():
            fire(j + 3, 1)

        return carry

    jax.lax.fori_loop(0, _QPW_B // 2, step, 0)
    pltpu.sync_copy(ob, out_ref.at[pl.ds(base, _QPW_B)])


def _topk_pallas(d2f, thf):
    f = functools.partial(
        pl.kernel,
        mesh=plsc.VectorSubcoreMesh(core_axis_name="c", subcore_axis_name="s"),
        compiler_params=pltpu.CompilerParams(needs_layout_passes=False),
        out_type=jax.ShapeDtypeStruct((L2 * N, K), jnp.int32),
        scratch_types=[
            pltpu.VMEM((_QPW_B, K), jnp.float32),
            pltpu.VMEM((L,), jnp.float32),
            pltpu.VMEM((L,), jnp.float32),
            pltpu.VMEM((_QPW_B, K), jnp.int32),
            pltpu.VMEM((K,), jnp.int32),
            pltpu.VMEM((K,), jnp.int32),
            pltpu.SemaphoreType.DMA,
            pltpu.SemaphoreType.DMA,
        ],
    )(_topk_body)
    return f(d2f, thf)


def _pool_body(cidx_ref, table_ref, out_ref, idx_v, buf0, buf1, ob, sem0, sem1):
    wid = jax.lax.axis_index("s") * _NC + jax.lax.axis_index("c")
    pltpu.sync_copy(cidx_ref.at[wid], idx_v)
    bufs = (buf0, buf1)
    sems = (sem0, sem1)

    def fire(c, b):
        pltpu.make_async_copy(table_ref.at[idx_v.at[c]], bufs[b], sems[b]).start()

    def wait(b):
        pltpu.make_async_copy(table_ref.at[idx_v.at[0]], bufs[b], sems[b]).wait()

    def compute(c, b):
        buf = bufs[b]

        def dstep(d, carry):
            off = d * 16
            for qq in range(_CH):
                m = buf[qq * K, pl.ds(off, 16)]
                for r in range(1, K):
                    m = jnp.maximum(m, buf[qq * K + r, pl.ds(off, 16)])
                ob[qq, pl.ds(off, 16)] = m
            return carry

        jax.lax.fori_loop(0, D_OUT // 16, dstep, 0)
        pltpu.sync_copy(ob, out_ref.at[pl.ds(wid * _QPW + c * _CH, _CH)])

    fire(0, 0)
    fire(1, 1)

    def step(i, carry):
        c = i * 2
        wait(0)
        compute(c, 0)

        @pl.when(c + 2 < _NCHUNK)
        def _():
            fire(c + 2, 0)

        wait(1)
        compute(c + 1, 1)

        @pl.when(c + 3 < _NCHUNK)
        def _():
            fire(c + 3, 1)

        return carry

    jax.lax.fori_loop(0, _NCHUNK // 2, step, 0)


def _pool_pallas(cidx3, table):
    f = functools.partial(
        pl.kernel,
        mesh=plsc.VectorSubcoreMesh(core_axis_name="c", subcore_axis_name="s"),
        out_type=jax.ShapeDtypeStruct((L2 * N, D_OUT), jnp.float32),
        scratch_types=[
            pltpu.VMEM((_NCHUNK, _ROWS), jnp.int32),
            pltpu.VMEM((_ROWS, D_OUT), jnp.float32),
            pltpu.VMEM((_ROWS, D_OUT), jnp.float32),
            pltpu.VMEM((_CH, D_OUT), jnp.float32),
            pltpu.SemaphoreType.DMA,
            pltpu.SemaphoreType.DMA,
        ],
    )(_pool_body)
    return f(cidx3, table)


def _mlp_body(x_ref, w1_ref, g1_ref, b1_ref, w2_ref, g2_ref, b2_ref, o_ref):
    x = x_ref[...]
    h = jnp.dot(x, w1_ref[...].T, preferred_element_type=jnp.float32)
    mu = jnp.mean(h, axis=-1, keepdims=True)
    var = jnp.mean((h - mu) ** 2, axis=-1, keepdims=True)
    h = (h - mu) / jnp.sqrt(var + 1e-5) * g1_ref[...] + b1_ref[...]
    h = jnp.dot(h, w2_ref[...].T, preferred_element_type=jnp.float32)
    mu = jnp.mean(h, axis=-1, keepdims=True)
    var = jnp.mean((h - mu) ** 2, axis=-1, keepdims=True)
    h = (h - mu) / jnp.sqrt(var + 1e-5) * g2_ref[...] + b2_ref[...]
    o_ref[...] = jnp.maximum(h, 0.0)


def _mlp_pallas(x2d, W1, g1, b1, W2, g2, b2):
    R = x2d.shape[0]
    BR = 1024
    grid = (R // BR,)
    return pl.pallas_call(
        _mlp_body,
        grid=grid,
        in_specs=[
            pl.BlockSpec((BR, D_IN), lambda i: (i, 0)),
            pl.BlockSpec((D_OUT, D_IN), lambda i: (0, 0)),
            pl.BlockSpec((D_OUT,), lambda i: (0,)),
            pl.BlockSpec((D_OUT,), lambda i: (0,)),
            pl.BlockSpec((D_OUT, D_OUT), lambda i: (0, 0)),
            pl.BlockSpec((D_OUT,), lambda i: (0,)),
            pl.BlockSpec((D_OUT,), lambda i: (0,)),
        ],
        out_specs=pl.BlockSpec((BR, D_OUT), lambda i: (i, 0)),
        out_shape=jax.ShapeDtypeStruct((R, D_OUT), jnp.float32),
    )(x2d, W1, g1, b1, W2, g2, b2)


def kernel(coords, features, W1, g1, b1, W2, g2, b2):
    keep = jax.random.permutation(jax.random.key(42), L)[:L2]
    keep_l = jnp.repeat(keep, N)
    keep_n = jnp.tile(jnp.arange(N), L2)
    keep_coords = coords[keep_l, keep_n].reshape(L2, N, C)

    x2d = features[:L2].reshape(L2 * N, D_IN)
    feats_sub = _mlp_pallas(x2d, W1, g1, b1, W2, g2, b2).reshape(L2, N, D_OUT)

    c1s = jnp.swapaxes(coords, 0, 1)
    c2s = jnp.swapaxes(keep_coords, 0, 1)
    d2 = (jnp.sum(c2s ** 2, axis=-1)[:, :, None]
          + jnp.sum(c1s ** 2, axis=-1)[:, None, :]
          - 2.0 * jnp.einsum('nqc,nlc->nql', c2s, c1s))
    gm = jnp.min(d2.reshape(N, L2, _NG, _GS), axis=-1)
    th = _theta_pallas(gm.reshape(N * L2, _NG))
    out_nq = _topk_pallas(d2.reshape(N * L2, L), th)
    out_qn = jnp.transpose(out_nq.reshape(N, L2, K),
                           (1, 0, 2)).reshape(L2 * N, K)

    idx0 = jnp.transpose(jnp.right_shift(out_qn, 3).reshape(L2, N, K),
                         (2, 0, 1)).reshape(-1)
    idx1 = jnp.broadcast_to(jnp.arange(N)[None, None, :], (K, L2, N)).reshape(-1)

    cidx3 = out_qn.reshape(_NW, _NCHUNK, _ROWS)
    table = feats_sub.reshape(L2 * N, D_OUT)
    pool = _pool_pallas(cidx3, table).reshape(L2, N, D_OUT)
    return keep_coords, pool, (idx0, idx1), (keep_l, keep_n)


# speedup vs baseline: 6.8475x; 1.0056x over previous
import functools

import jax
import jax.numpy as jnp
from jax.experimental import pallas as pl
from jax.experimental.pallas import tpu as pltpu
from jax.experimental.pallas import tpu_sc as plsc

L = 4096
N = 8
C = 3
D_IN = 128
D_OUT = 256
K = 16
L2 = 1024

_NC, _NS = 2, 16
_NW = _NC * _NS
_QPW = (L2 * N) // _NW
_CH = 4
_NCHUNK = _QPW // _CH
_ROWS = _CH * K


_BQ = 256
_NG = 128
_GS = 32


def _theta_body(gm_ref, th_ref):
    gm = gm_ref[...]
    lanes = jax.lax.broadcasted_iota(jnp.int32, (_BQ, _NG), 1).astype(jnp.float32)
    for t in range(K - 1):
        m = jnp.min(gm, axis=1, keepdims=True)
        gid = jnp.min(jnp.where(gm == m, lanes, 1e9), axis=1, keepdims=True)
        gm = jnp.where(lanes == gid, jnp.inf, gm)
    th = jnp.min(gm, axis=1, keepdims=True)
    th_ref[...] = jnp.broadcast_to(th, (_BQ, K))


def _theta_pallas(gm):
    return pl.pallas_call(
        _theta_body,
        grid=((N * L2) // _BQ,),
        in_specs=[pl.BlockSpec((_BQ, _NG), lambda i: (i, 0))],
        out_specs=pl.BlockSpec((_BQ, K), lambda i: (i, 0)),
        out_shape=jax.ShapeDtypeStruct((N * L2, K), jnp.float32),
    )(gm)


_QPW_B = (L2 * N) // _NW


def _topk_body(d_ref, th_ref, out_ref, th_v, dv0, dv1, ob, tvb, tib,
               sem0, sem1):
    wid = jax.lax.axis_index("s") * _NC + jax.lax.axis_index("c")
    base = wid * _QPW_B
    pltpu.sync_copy(th_ref.at[pl.ds(base, _QPW_B)], th_v)
    dbufs = (dv0, dv1)
    sems = (sem0, sem1)

    def fire(j, b):
        pltpu.make_async_copy(d_ref.at[base + j], dbufs[b], sems[b]).start()

    def wait(b):
        pltpu.make_async_copy(d_ref.at[base], dbufs[b], sems[b]).wait()

    iota16 = jax.lax.iota(jnp.int32, 16)

    def compute(j, b):
        dv = dbufs[b]
        th = th_v[j, pl.ds(0, 16)]
        nn = jax.lax.shift_right_logical(base + j, 10)
        tvb[pl.ds(0, 16)] = jnp.full((16,), 0x7F800000, dtype=jnp.int32)
        tib[pl.ds(0, 16)] = jnp.zeros((16,), dtype=jnp.int32)

        def merge(val, idxv):
            bits = plsc.bitcast(val, jnp.int32)
            key = jnp.where(val < 0.0,
                            jnp.bitwise_xor(bits, 0x7FFFFFFF), bits)
            sv, si = plsc.sort_key_val(key, idxv, descending=True)
            tvv = tvb[pl.ds(0, 16)]
            tii = tib[pl.ds(0, 16)]
            take = sv < tvv
            mv = jnp.where(take, sv, tvv)
            mi = jnp.where(take, si, tii)
            tv2, ti2 = plsc.sort_key_val(mv, mi)
            tvb[pl.ds(0, 16)] = tv2
            tib[pl.ds(0, 16)] = ti2

        _U = 8

        def chunk(c4, carry):
            cbase = c4 * _U
            vals = [dv[pl.ds((cbase + u) * 16, 16)] for u in range(_U)]
            hits = [v <= th for v in vals]
            anyv = hits[0]
            for u in range(1, _U):
                anyv = jnp.logical_or(anyv, hits[u])
            cnt = plsc.all_reduce_population_count(anyv)

            @pl.when(cnt[0] > 0)
            def _():
                for u in range(_U):
                    cntu = plsc.all_reduce_population_count(hits[u])

                    @pl.when(cntu[0] > 0)
                    def _(u=u):
                        merge(vals[u], iota16 + (cbase + u) * 16)

            return carry

        jax.lax.fori_loop(0, L // 16 // _U, chunk, 0)
        ti = tib[pl.ds(0, 16)]
        ob[j, pl.ds(0, 16)] = (
            jnp.left_shift(jnp.bitwise_and(ti, L2 - 1), 3) + nn)

    fire(0, 0)
    fire(1, 1)

    def step(i, carry):
        j = i * 2
        wait(0)
        compute(j, 0)

        @pl.when(j + 2 < _QPW_B)
        def _():
            fire(j + 2, 0)

        wait(1)
        compute(j + 1, 1)

        @pl.when(j + 3 < _QPW_B)
        def _():
            fire(j + 3, 1)

        return carry

    jax.lax.fori_loop(0, _QPW_B // 2, step, 0)
    pltpu.sync_copy(ob, out_ref.at[pl.ds(base, _QPW_B)])


def _topk_pallas(d2f, thf):
    f = functools.partial(
        pl.kernel,
        mesh=plsc.VectorSubcoreMesh(core_axis_name="c", subcore_axis_name="s"),
        compiler_params=pltpu.CompilerParams(needs_layout_passes=False),
        out_type=jax.ShapeDtypeStruct((L2 * N, K), jnp.int32),
        scratch_types=[
            pltpu.VMEM((_QPW_B, K), jnp.float32),
            pltpu.VMEM((L,), jnp.float32),
            pltpu.VMEM((L,), jnp.float32),
            pltpu.VMEM((_QPW_B, K), jnp.int32),
            pltpu.VMEM((K,), jnp.int32),
            pltpu.VMEM((K,), jnp.int32),
            pltpu.SemaphoreType.DMA,
            pltpu.SemaphoreType.DMA,
        ],
    )(_topk_body)
    return f(d2f, thf)


def _pool_body(cidx_ref, table_ref, out_ref, idx_v, buf0, buf1, ob, sem0, sem1):
    wid = jax.lax.axis_index("s") * _NC + jax.lax.axis_index("c")
    pltpu.sync_copy(cidx_ref.at[wid], idx_v)
    bufs = (buf0, buf1)
    sems = (sem0, sem1)

    def fire(c, b):
        pltpu.make_async_copy(table_ref.at[idx_v.at[c]], bufs[b], sems[b]).start()

    def wait(b):
        pltpu.make_async_copy(table_ref.at[idx_v.at[0]], bufs[b], sems[b]).wait()

    def compute(c, b):
        buf = bufs[b]

        def dstep(d, carry):
            off = d * 16
            for qq in range(_CH):
                m = buf[qq * K, pl.ds(off, 16)]
                for r in range(1, K):
                    m = jnp.maximum(m, buf[qq * K + r, pl.ds(off, 16)])
                ob[qq, pl.ds(off, 16)] = m
            return carry

        jax.lax.fori_loop(0, D_OUT // 16, dstep, 0)
        pltpu.sync_copy(ob, out_ref.at[pl.ds(wid * _QPW + c * _CH, _CH)])

    fire(0, 0)
    fire(1, 1)

    def step(i, carry):
        c = i * 2
        wait(0)
        compute(c, 0)

        @pl.when(c + 2 < _NCHUNK)
        def _():
            fire(c + 2, 0)

        wait(1)
        compute(c + 1, 1)

        @pl.when(c + 3 < _NCHUNK)
        def _():
            fire(c + 3, 1)

        return carry

    jax.lax.fori_loop(0, _NCHUNK // 2, step, 0)


def _pool_pallas(cidx3, table):
    f = functools.partial(
        pl.kernel,
        mesh=plsc.VectorSubcoreMesh(core_axis_name="c", subcore_axis_name="s"),
        out_type=jax.ShapeDtypeStruct((L2 * N, D_OUT), jnp.float32),
        scratch_types=[
            pltpu.VMEM((_NCHUNK, _ROWS), jnp.int32),
            pltpu.VMEM((_ROWS, D_OUT), jnp.float32),
            pltpu.VMEM((_ROWS, D_OUT), jnp.float32),
            pltpu.VMEM((_CH, D_OUT), jnp.float32),
            pltpu.SemaphoreType.DMA,
            pltpu.SemaphoreType.DMA,
        ],
    )(_pool_body)
    return f(cidx3, table)


def _mlp_body(x_ref, w1_ref, g1_ref, b1_ref, w2_ref, g2_ref, b2_ref, o_ref):
    x = x_ref[...]
    h = jnp.dot(x, w1_ref[...].T, preferred_element_type=jnp.float32)
    mu = jnp.mean(h, axis=-1, keepdims=True)
    var = jnp.mean((h - mu) ** 2, axis=-1, keepdims=True)
    h = (h - mu) / jnp.sqrt(var + 1e-5) * g1_ref[...] + b1_ref[...]
    h = jnp.dot(h, w2_ref[...].T, preferred_element_type=jnp.float32)
    mu = jnp.mean(h, axis=-1, keepdims=True)
    var = jnp.mean((h - mu) ** 2, axis=-1, keepdims=True)
    h = (h - mu) / jnp.sqrt(var + 1e-5) * g2_ref[...] + b2_ref[...]
    o_ref[...] = jnp.maximum(h, 0.0)


def _mlp_pallas(x2d, W1, g1, b1, W2, g2, b2):
    R = x2d.shape[0]
    BR = 1024
    grid = (R // BR,)
    return pl.pallas_call(
        _mlp_body,
        grid=grid,
        in_specs=[
            pl.BlockSpec((BR, D_IN), lambda i: (i, 0)),
            pl.BlockSpec((D_OUT, D_IN), lambda i: (0, 0)),
            pl.BlockSpec((D_OUT,), lambda i: (0,)),
            pl.BlockSpec((D_OUT,), lambda i: (0,)),
            pl.BlockSpec((D_OUT, D_OUT), lambda i: (0, 0)),
            pl.BlockSpec((D_OUT,), lambda i: (0,)),
            pl.BlockSpec((D_OUT,), lambda i: (0,)),
        ],
        out_specs=pl.BlockSpec((BR, D_OUT), lambda i: (i, 0)),
        out_shape=jax.ShapeDtypeStruct((R, D_OUT), jnp.float32),
    )(x2d, W1, g1, b1, W2, g2, b2)


def kernel(coords, features, W1, g1, b1, W2, g2, b2):
    keep = jax.random.permutation(jax.random.key(42), L)[:L2]
    keep_l = jnp.repeat(keep, N)
    keep_n = jnp.tile(jnp.arange(N), L2)
    keep_coords = coords[keep_l, keep_n].reshape(L2, N, C)

    x2d = features[:L2].reshape(L2 * N, D_IN)
    feats_sub = _mlp_pallas(x2d, W1, g1, b1, W2, g2, b2).reshape(L2, N, D_OUT)

    c1s = jnp.swapaxes(coords, 0, 1)
    c2s = jnp.swapaxes(keep_coords, 0, 1)
    d2 = (jnp.sum(c2s ** 2, axis=-1)[:, :, None]
          + jnp.sum(c1s ** 2, axis=-1)[:, None, :]
          - 2.0 * jnp.einsum('nqc,nlc->nql', c2s, c1s))
    gm = jnp.min(d2.reshape(N, L2, _NG, _GS), axis=-1)
    th = _theta_pallas(gm.reshape(N * L2, _NG))
    out_nq = _topk_pallas(d2.reshape(N * L2, L), th)
    out_qn = jnp.transpose(out_nq.reshape(N, L2, K),
                           (1, 0, 2)).reshape(L2 * N, K)

    idx0 = jnp.transpose(jnp.right_shift(out_qn, 3).reshape(L2, N, K),
                         (2, 0, 1)).reshape(-1)
    idx1 = jnp.broadcast_to(jnp.arange(N)[None, None, :], (K, L2, N)).reshape(-1)

    cidx3 = out_qn.reshape(_NW, _NCHUNK, _ROWS)
    table = feats_sub.reshape(L2 * N, D_OUT)
    pool = _pool_pallas(cidx3, table).reshape(L2, N, D_OUT)
    return keep_coords, pool, (idx0, idx1), (keep_l, keep_n)
